# baseline (speedup 1.0000x reference)
import functools

import jax
import jax.numpy as jnp
from jax import lax
from jax.experimental import pallas as pl
from jax.experimental.pallas import tpu as pltpu
from jax.experimental.pallas import tpu_sc as plsc

NC = 2
NS = 16
CHUNK = 80
BOUNCE = 128


def _npad(n):
  q = NS * 8
  return ((n + q - 1) // q) * q


def _sc_deg(dst3, zeros_np, n, e):
  nw = NC * NS
  assert e % (nw * CHUNK) == 0
  kper = e // (nw * CHUNK)
  npad = _npad(n)

  @functools.partial(
      pl.kernel,
      mesh=plsc.VectorSubcoreMesh(core_axis_name="c", subcore_axis_name="s"),
      out_type=jax.ShapeDtypeStruct((NC * npad,), jnp.float32),
      scratch_types=[
          pltpu.VMEM((kper, CHUNK), jnp.int32),
          pltpu.VMEM((CHUNK,), jnp.float32),
          pltpu.VMEM((npad,), jnp.float32),
          pltpu.VMEM_SHARED((npad,), jnp.float32),
          pltpu.SemaphoreType.DMA,
          pltpu.SemaphoreType.DMA,
      ],
  )
  def body(dst_hbm, z_hbm, out_hbm, didx, ones, dbuf, acc, lsem, ssem):
    cid = lax.axis_index("c")
    sid = lax.axis_index("s")
    w = cid * NS + sid

    @pl.when(sid == 0)
    def _():
      pltpu.sync_copy(z_hbm, dbuf)
      pltpu.sync_copy(dbuf, acc)
    for j in range(CHUNK // 16):
      ones[pl.ds(j * 16, 16)] = jnp.full((16,), 1.0, jnp.float32)
    pltpu.async_copy(dst_hbm.at[w], didx, lsem).wait()
    plsc.subcore_barrier()

    def step(k, _):
      pltpu.sync_copy(ones, acc.at[didx.at[k]], add=True)
      return 0

    lax.fori_loop(0, kper, step, 0)
    plsc.subcore_barrier()

    @pl.when(sid == 0)
    def _():
      pltpu.sync_copy(acc, dbuf)
      pltpu.sync_copy(dbuf, out_hbm.at[pl.ds(cid * npad, npad)])

  return body(dst3, zeros_np)


NB = 8
CHUNK2 = 128


@functools.lru_cache(maxsize=None)
def _sc_scatter_kernel(n, d, e):
  nw = NC * NS
  dh = d // 2
  assert e % (NS * CHUNK2) == 0
  kper = e // (NS * CHUNK2)
  assert kper % NB == 0
  npad = _npad(n)
  rper = npad // NS
  bnc = rper // 4
  nb = 4

  @functools.partial(
      pl.kernel,
      mesh=plsc.VectorSubcoreMesh(core_axis_name="c", subcore_axis_name="s"),
      out_type=jax.ShapeDtypeStruct((NC, npad, dh), jnp.float32),
      compiler_params=pltpu.CompilerParams(use_tc_tiling_on_sc=False),
      scratch_types=[
          pltpu.VMEM((NB, CHUNK2), jnp.int32),
          pltpu.VMEM((NB, CHUNK2), jnp.int32),
          [pltpu.VMEM((CHUNK2, dh), jnp.float32) for _ in range(NB)],
          pltpu.VMEM((bnc, dh), jnp.float32),
          pltpu.VMEM_SHARED((npad, dh), jnp.float32),
          pltpu.SemaphoreType.DMA,
          pltpu.SemaphoreType.DMA,
          pltpu.SemaphoreType.DMA,
      ],
  )
  def body(yl_hbm, yr_hbm, src_hbm, dst_hbm, z_hbm, out_hbm, sbuf, dbuf,
           rows, zbuf, acc, isem, gsem, ssem):
    cid = lax.axis_index("c")
    sid = lax.axis_index("s")

    pltpu.sync_copy(z_hbm, zbuf)
    for i in range(nb):
      pltpu.sync_copy(zbuf, acc.at[pl.ds(sid * rper + i * bnc, bnc)])
    plsc.subcore_barrier()

    def group_body(y_hbm):
      def group(g, _):
        pltpu.async_copy(src_hbm.at[sid, g], sbuf, isem)
        pltpu.async_copy(dst_hbm.at[sid, g], dbuf, isem)
        pltpu.make_async_copy(src_hbm.at[sid, 0], sbuf, isem).wait()
        pltpu.make_async_copy(dst_hbm.at[sid, 0], dbuf, isem).wait()
        for b in range(NB):
          pltpu.async_copy(y_hbm.at[sbuf.at[b]], rows[b], gsem)
        for b in range(NB):
          pltpu.make_async_copy(y_hbm.at[sbuf.at[b]], rows[b], gsem).wait()
          pltpu.async_copy(rows[b], acc.at[dbuf.at[b]], ssem, add=True)
        for b in range(NB):
          pltpu.make_async_copy(rows[b], acc.at[dbuf.at[b]], ssem).wait()
        return 0
      return group

    @pl.when(cid == 0)
    def _():
      lax.fori_loop(0, kper // NB, group_body(yl_hbm), 0)

    @pl.when(cid == 1)
    def _():
      lax.fori_loop(0, kper // NB, group_body(yr_hbm), 0)

    plsc.subcore_barrier()
    for i in range(nb):
      r0 = sid * rper + i * bnc
      pltpu.sync_copy(acc.at[pl.ds(r0, bnc)], zbuf)
      pltpu.sync_copy(zbuf, out_hbm.at[cid, pl.ds(r0, bnc)])

  return body


def _sc_scatter(yl, yr, src3, dst3, zeros_bd, n, d, e):
  return _sc_scatter_kernel(n, d, e)(yl, yr, src3, dst3, zeros_bd)


def _tc_h0_body(x_ref, w_ref, b_ref, o_ref):
  h = jnp.dot(x_ref[...], w_ref[...], preferred_element_type=jnp.float32)
  o_ref[...] = jnp.maximum(h + b_ref[...], 0.0)


def _dinv_col(deg_ref, n):
  d = lax.rsqrt(deg_ref[0, :n] + deg_ref[1, :n] + 1.0)
  return d.reshape(n, 1)


def _tc_y_body(n, h_ref, w_ref, deg_ref, ol_ref, or_ref):
  dh = ol_ref.shape[1]
  dinv = _dinv_col(deg_ref, n)
  xw = jnp.dot(h_ref[...], w_ref[...], preferred_element_type=jnp.float32)
  y = xw * dinv
  ol_ref[...] = y[:, :dh]
  or_ref[...] = y[:, dh:]


def _tc_layer_body(n, p_ref, yl_ref, yr_ref, deg_ref, b_ref, w_ref,
                   ol_ref, or_ref):
  dh = ol_ref.shape[1]
  dinv = _dinv_col(deg_ref, n)
  p = jnp.concatenate([p_ref[0, :n], p_ref[1, :n]], axis=1)
  y = jnp.concatenate([yl_ref[...], yr_ref[...]], axis=1)
  h = jnp.maximum(dinv * (p + y) + b_ref[...], 0.0)
  xw = jnp.dot(h, w_ref[...], preferred_element_type=jnp.float32)
  yn = xw * dinv
  ol_ref[...] = yn[:, :dh]
  or_ref[...] = yn[:, dh:]


def _tc_final_body(n, p_ref, yl_ref, yr_ref, deg_ref, b_ref, g_ref, bt_ref,
                   o_ref):
  dinv = _dinv_col(deg_ref, n)
  p = jnp.concatenate([p_ref[0, :n], p_ref[1, :n]], axis=1)
  y = jnp.concatenate([yl_ref[...], yr_ref[...]], axis=1)
  h = jnp.maximum(dinv * (p + y) + b_ref[...], 0.0)
  mean = jnp.mean(h, axis=0, keepdims=True)
  cen = h - mean
  var = jnp.mean(cen * cen, axis=0, keepdims=True)
  o_ref[...] = cen * lax.rsqrt(var + 1e-5) * g_ref[...] + bt_ref[...]


def _tc_call(body, out_shape, *args):
  return pl.pallas_call(body, out_shape=out_shape)(*args)


def kernel(x, edge_index, batch, W0, b0, Wc1, bc1, Wc2, bc2, Wc3, bc3,
           gamma, beta):
  n, d = x.shape
  dh = d // 2
  e = edge_index.shape[1]
  npad = _npad(n)
  nw = NC * NS
  kper_w = e // (nw * CHUNK)
  dst3w = edge_index[1].reshape(nw, kper_w, CHUNK)
  q = NB * CHUNK2
  perp = -(-(e // NS) // q) * q
  ep = NS * perp
  npad_ids = jnp.arange(ep - e, dtype=jnp.int32)
  src_p = jnp.concatenate([edge_index[0], npad_ids % n])
  dst_p = jnp.concatenate([edge_index[1], n + npad_ids % (npad - n)])
  src3 = src_p.reshape(NS, perp // q, NB, CHUNK2)
  dst3 = dst_p.reshape(NS, perp // q, NB, CHUNK2)
  zeros_np = jnp.zeros((npad,), jnp.float32)
  zeros_bd = jnp.zeros((npad // NS // 4, dh), jnp.float32)
  f32 = jnp.float32

  degp = _sc_deg(dst3w, zeros_np, n, e).reshape(NC, npad)
  h0 = _tc_call(_tc_h0_body, jax.ShapeDtypeStruct((n, d), f32),
                x, W0, b0.reshape(1, d))
  yhalf = jax.ShapeDtypeStruct((n, dh), f32)
  yl, yr = _tc_call(functools.partial(_tc_y_body, n), [yhalf, yhalf],
                    h0, Wc1, degp)

  for (w_next, b_cur) in ((Wc2, bc1), (Wc3, bc2)):
    p = _sc_scatter(yl, yr, src3, dst3, zeros_bd, n, d, ep)
    yl, yr = _tc_call(functools.partial(_tc_layer_body, n), [yhalf, yhalf],
                      p, yl, yr, degp, b_cur.reshape(1, d), w_next)

  p = _sc_scatter(yl, yr, src3, dst3, zeros_bd, n, d, ep)
  out = _tc_call(functools.partial(_tc_final_body, n),
                 jax.ShapeDtypeStruct((n, d), f32),
                 p, yl, yr, degp, bc3.reshape(1, d),
                 gamma.reshape(1, d), beta.reshape(1, d))
  return out


# speedup vs baseline: 1.0017x; 1.0017x over previous
import functools

import jax
import jax.numpy as jnp
from jax import lax
from jax.experimental import pallas as pl
from jax.experimental.pallas import tpu as pltpu
from jax.experimental.pallas import tpu_sc as plsc

NC = 2
NS = 16
CHUNK = 80
BOUNCE = 128


def _npad(n):
  q = NS * 8
  return ((n + q - 1) // q) * q


def _sc_deg(dst3, zeros_np, n, e):
  nw = NC * NS
  assert e % (nw * CHUNK) == 0
  kper = e // (nw * CHUNK)
  npad = _npad(n)

  @functools.partial(
      pl.kernel,
      mesh=plsc.VectorSubcoreMesh(core_axis_name="c", subcore_axis_name="s"),
      out_type=jax.ShapeDtypeStruct((NC * npad,), jnp.float32),
      scratch_types=[
          pltpu.VMEM((kper, CHUNK), jnp.int32),
          pltpu.VMEM((CHUNK,), jnp.float32),
          pltpu.VMEM((npad,), jnp.float32),
          pltpu.VMEM_SHARED((npad,), jnp.float32),
          pltpu.SemaphoreType.DMA,
          pltpu.SemaphoreType.DMA,
      ],
  )
  def body(dst_hbm, z_hbm, out_hbm, didx, ones, dbuf, acc, lsem, ssem):
    cid = lax.axis_index("c")
    sid = lax.axis_index("s")
    w = cid * NS + sid

    @pl.when(sid == 0)
    def _():
      pltpu.sync_copy(z_hbm, dbuf)
      pltpu.sync_copy(dbuf, acc)
    for j in range(CHUNK // 16):
      ones[pl.ds(j * 16, 16)] = jnp.full((16,), 1.0, jnp.float32)
    pltpu.async_copy(dst_hbm.at[w], didx, lsem).wait()
    plsc.subcore_barrier()

    def step(k, _):
      pltpu.sync_copy(ones, acc.at[didx.at[k]], add=True)
      return 0

    lax.fori_loop(0, kper, step, 0)
    plsc.subcore_barrier()

    @pl.when(sid == 0)
    def _():
      pltpu.sync_copy(acc, dbuf)
      pltpu.sync_copy(dbuf, out_hbm.at[pl.ds(cid * npad, npad)])

  return body(dst3, zeros_np)


NB = 8
CHUNK2 = 128


@functools.lru_cache(maxsize=None)
def _sc_scatter_kernel(n, d, e):
  nw = NC * NS
  dh = d // 2
  assert e % (NS * CHUNK2) == 0
  kper = e // (NS * CHUNK2)
  assert kper % NB == 0
  npad = _npad(n)
  rper = npad // NS
  bnc = rper // 4
  nb = 4

  @functools.partial(
      pl.kernel,
      mesh=plsc.VectorSubcoreMesh(core_axis_name="c", subcore_axis_name="s"),
      out_type=jax.ShapeDtypeStruct((NC, npad, dh), jnp.float32),
      compiler_params=pltpu.CompilerParams(use_tc_tiling_on_sc=False),
      scratch_types=[
          pltpu.VMEM((NB, CHUNK2), jnp.int32),
          pltpu.VMEM((NB, CHUNK2), jnp.int32),
          [pltpu.VMEM((CHUNK2, dh), jnp.float32) for _ in range(NB)],
          pltpu.VMEM((bnc, dh), jnp.float32),
          pltpu.VMEM_SHARED((npad, dh), jnp.float32),
          pltpu.SemaphoreType.DMA,
          pltpu.SemaphoreType.DMA,
          pltpu.SemaphoreType.DMA,
      ],
  )
  def body(yl_hbm, yr_hbm, src_hbm, dst_hbm, z_hbm, out_hbm, sbuf, dbuf,
           rows, zbuf, acc, isem, gsem, ssem):
    cid = lax.axis_index("c")
    sid = lax.axis_index("s")

    pltpu.sync_copy(z_hbm, zbuf)
    for i in range(nb):
      pltpu.sync_copy(zbuf, acc.at[pl.ds(sid * rper + i * bnc, bnc)])
    plsc.subcore_barrier()

    def group_body(y_hbm):
      def group(g, _):
        pltpu.async_copy(src_hbm.at[sid, g], sbuf, isem)
        pltpu.async_copy(dst_hbm.at[sid, g], dbuf, isem)
        pltpu.make_async_copy(src_hbm.at[sid, 0], sbuf, isem).wait()
        pltpu.make_async_copy(dst_hbm.at[sid, 0], dbuf, isem).wait()
        for b in range(NB):
          pltpu.async_copy(y_hbm.at[sbuf.at[b]], rows[b], gsem)
        for b in range(NB):
          pltpu.make_async_copy(y_hbm.at[sbuf.at[b]], rows[b], gsem).wait()
          pltpu.async_copy(rows[b], acc.at[dbuf.at[b]], ssem, add=True)
        for b in range(NB):
          pltpu.make_async_copy(rows[b], acc.at[dbuf.at[b]], ssem).wait()
        return 0
      return group

    @pl.when(cid == 0)
    def _():
      lax.fori_loop(0, kper // NB, group_body(yl_hbm), 0)

    @pl.when(cid == 1)
    def _():
      lax.fori_loop(0, kper // NB, group_body(yr_hbm), 0)

    plsc.subcore_barrier()
    for i in range(nb):
      r0 = sid * rper + i * bnc
      pltpu.sync_copy(acc.at[pl.ds(r0, bnc)], zbuf)
      pltpu.sync_copy(zbuf, out_hbm.at[cid, pl.ds(r0, bnc)])

  return body


def _sc_scatter(yl, yr, src3, dst3, zeros_bd, n, d, e):
  return _sc_scatter_kernel(n, d, e)(yl, yr, src3, dst3, zeros_bd)


def _dinv_col(deg_ref, n):
  d = lax.rsqrt(deg_ref[0, :n] + deg_ref[1, :n] + 1.0)
  return d.reshape(n, 1)


def _tc_y_body(n, x_ref, w0_ref, b0_ref, w_ref, deg_ref, ol_ref, or_ref):
  dh = ol_ref.shape[1]
  dinv = _dinv_col(deg_ref, n)
  h = jnp.maximum(
      jnp.dot(x_ref[...], w0_ref[...], preferred_element_type=jnp.float32)
      + b0_ref[...], 0.0)
  xw = jnp.dot(h, w_ref[...], preferred_element_type=jnp.float32)
  y = xw * dinv
  ol_ref[...] = y[:, :dh]
  or_ref[...] = y[:, dh:]


def _tc_layer_body(n, p_ref, yl_ref, yr_ref, deg_ref, b_ref, w_ref,
                   ol_ref, or_ref):
  dh = ol_ref.shape[1]
  dinv = _dinv_col(deg_ref, n)
  p = jnp.concatenate([p_ref[0, :n], p_ref[1, :n]], axis=1)
  y = jnp.concatenate([yl_ref[...], yr_ref[...]], axis=1)
  h = jnp.maximum(dinv * (p + y) + b_ref[...], 0.0)
  xw = jnp.dot(h, w_ref[...], preferred_element_type=jnp.float32)
  yn = xw * dinv
  ol_ref[...] = yn[:, :dh]
  or_ref[...] = yn[:, dh:]


def _tc_final_body(n, p_ref, yl_ref, yr_ref, deg_ref, b_ref, g_ref, bt_ref,
                   o_ref):
  dinv = _dinv_col(deg_ref, n)
  p = jnp.concatenate([p_ref[0, :n], p_ref[1, :n]], axis=1)
  y = jnp.concatenate([yl_ref[...], yr_ref[...]], axis=1)
  h = jnp.maximum(dinv * (p + y) + b_ref[...], 0.0)
  mean = jnp.mean(h, axis=0, keepdims=True)
  cen = h - mean
  var = jnp.mean(cen * cen, axis=0, keepdims=True)
  o_ref[...] = cen * lax.rsqrt(var + 1e-5) * g_ref[...] + bt_ref[...]


def _tc_call(body, out_shape, *args):
  return pl.pallas_call(body, out_shape=out_shape)(*args)


def kernel(x, edge_index, batch, W0, b0, Wc1, bc1, Wc2, bc2, Wc3, bc3,
           gamma, beta):
  n, d = x.shape
  dh = d // 2
  e = edge_index.shape[1]
  npad = _npad(n)
  nw = NC * NS
  kper_w = e // (nw * CHUNK)
  dst3w = edge_index[1].reshape(nw, kper_w, CHUNK)
  q = NB * CHUNK2
  perp = -(-(e // NS) // q) * q
  ep = NS * perp
  npad_ids = jnp.arange(ep - e, dtype=jnp.int32)
  src_p = jnp.concatenate([edge_index[0], npad_ids % n])
  dst_p = jnp.concatenate([edge_index[1], n + npad_ids % (npad - n)])
  src3 = src_p.reshape(NS, perp // q, NB, CHUNK2)
  dst3 = dst_p.reshape(NS, perp // q, NB, CHUNK2)
  zeros_np = jnp.zeros((npad,), jnp.float32)
  zeros_bd = jnp.zeros((npad // NS // 4, dh), jnp.float32)
  f32 = jnp.float32

  degp = _sc_deg(dst3w, zeros_np, n, e).reshape(NC, npad)
  yhalf = jax.ShapeDtypeStruct((n, dh), f32)
  yl, yr = _tc_call(functools.partial(_tc_y_body, n), [yhalf, yhalf],
                    x, W0, b0.reshape(1, d), Wc1, degp)

  for (w_next, b_cur) in ((Wc2, bc1), (Wc3, bc2)):
    p = _sc_scatter(yl, yr, src3, dst3, zeros_bd, n, d, ep)
    yl, yr = _tc_call(functools.partial(_tc_layer_body, n), [yhalf, yhalf],
                      p, yl, yr, degp, b_cur.reshape(1, d), w_next)

  p = _sc_scatter(yl, yr, src3, dst3, zeros_bd, n, d, ep)
  out = _tc_call(functools.partial(_tc_final_body, n),
                 jax.ShapeDtypeStruct((n, d), f32),
                 p, yl, yr, degp, bc3.reshape(1, d),
                 gamma.reshape(1, d), beta.reshape(1, d))
  return out


# speedup vs baseline: 1.0112x; 1.0095x over previous
import functools

import jax
import jax.numpy as jnp
from jax import lax
from jax.experimental import pallas as pl
from jax.experimental.pallas import tpu as pltpu
from jax.experimental.pallas import tpu_sc as plsc

NC = 2
NS = 16
CHUNK = 80
BOUNCE = 128


def _npad(n):
  q = NS * 8
  return ((n + q - 1) // q) * q


def _sc_deg(dst3, zeros_np, n, e):
  nw = NC * NS
  assert e % (nw * CHUNK) == 0
  kper = e // (nw * CHUNK)
  npad = _npad(n)

  @functools.partial(
      pl.kernel,
      mesh=plsc.VectorSubcoreMesh(core_axis_name="c", subcore_axis_name="s"),
      out_type=jax.ShapeDtypeStruct((NC * npad,), jnp.float32),
      scratch_types=[
          pltpu.VMEM((kper, CHUNK), jnp.int32),
          pltpu.VMEM((CHUNK,), jnp.float32),
          pltpu.VMEM((npad,), jnp.float32),
          pltpu.VMEM_SHARED((npad,), jnp.float32),
          pltpu.SemaphoreType.DMA,
          pltpu.SemaphoreType.DMA,
      ],
  )
  def body(dst_hbm, z_hbm, out_hbm, didx, ones, dbuf, acc, lsem, ssem):
    cid = lax.axis_index("c")
    sid = lax.axis_index("s")
    w = cid * NS + sid

    @pl.when(sid == 0)
    def _():
      pltpu.sync_copy(z_hbm, dbuf)
      pltpu.sync_copy(dbuf, acc)
    for j in range(CHUNK // 16):
      ones[pl.ds(j * 16, 16)] = jnp.full((16,), 1.0, jnp.float32)
    pltpu.async_copy(dst_hbm.at[w], didx, lsem).wait()
    plsc.subcore_barrier()

    def step(k, _):
      pltpu.sync_copy(ones, acc.at[didx.at[k]], add=True)
      return 0

    lax.fori_loop(0, kper, step, 0)
    plsc.subcore_barrier()

    @pl.when(sid == 0)
    def _():
      pltpu.sync_copy(acc, dbuf)
      pltpu.sync_copy(dbuf, out_hbm.at[pl.ds(cid * npad, npad)])

  return body(dst3, zeros_np)


NB = 12
CHUNK2 = 80


@functools.lru_cache(maxsize=None)
def _sc_scatter_kernel(n, d, e):
  nw = NC * NS
  dh = d // 2
  assert e % (NS * CHUNK2) == 0
  kper = e // (NS * CHUNK2)
  assert kper % NB == 0
  npad = _npad(n)
  rper = npad // NS
  bnc = rper // 4
  nb = 4

  @functools.partial(
      pl.kernel,
      mesh=plsc.VectorSubcoreMesh(core_axis_name="c", subcore_axis_name="s"),
      out_type=jax.ShapeDtypeStruct((NC, npad, dh), jnp.float32),
      compiler_params=pltpu.CompilerParams(use_tc_tiling_on_sc=False),
      scratch_types=[
          pltpu.VMEM((NB, CHUNK2), jnp.int32),
          pltpu.VMEM((NB, CHUNK2), jnp.int32),
          [pltpu.VMEM((CHUNK2, dh), jnp.float32) for _ in range(NB)],
          pltpu.VMEM((bnc, dh), jnp.float32),
          pltpu.VMEM_SHARED((npad, dh), jnp.float32),
          pltpu.SemaphoreType.DMA,
          pltpu.SemaphoreType.DMA,
          pltpu.SemaphoreType.DMA,
      ],
  )
  def body(yl_hbm, yr_hbm, src_hbm, dst_hbm, z_hbm, out_hbm, sbuf, dbuf,
           rows, zbuf, acc, isem, gsem, ssem):
    cid = lax.axis_index("c")
    sid = lax.axis_index("s")

    pltpu.sync_copy(z_hbm, zbuf)
    for i in range(nb):
      pltpu.sync_copy(zbuf, acc.at[pl.ds(sid * rper + i * bnc, bnc)])
    plsc.subcore_barrier()

    def group_body(y_hbm):
      def group(g, _):
        pltpu.async_copy(src_hbm.at[sid, g], sbuf, isem)
        pltpu.async_copy(dst_hbm.at[sid, g], dbuf, isem)
        pltpu.make_async_copy(src_hbm.at[sid, 0], sbuf, isem).wait()
        pltpu.make_async_copy(dst_hbm.at[sid, 0], dbuf, isem).wait()
        for b in range(NB):
          pltpu.async_copy(y_hbm.at[sbuf.at[b]], rows[b], gsem)
        for b in range(NB):
          pltpu.make_async_copy(y_hbm.at[sbuf.at[b]], rows[b], gsem).wait()
          pltpu.async_copy(rows[b], acc.at[dbuf.at[b]], ssem, add=True)
        for b in range(NB):
          pltpu.make_async_copy(rows[b], acc.at[dbuf.at[b]], ssem).wait()
        return 0
      return group

    @pl.when(cid == 0)
    def _():
      lax.fori_loop(0, kper // NB, group_body(yl_hbm), 0)

    @pl.when(cid == 1)
    def _():
      lax.fori_loop(0, kper // NB, group_body(yr_hbm), 0)

    plsc.subcore_barrier()
    for i in range(nb):
      r0 = sid * rper + i * bnc
      pltpu.sync_copy(acc.at[pl.ds(r0, bnc)], zbuf)
      pltpu.sync_copy(zbuf, out_hbm.at[cid, pl.ds(r0, bnc)])

  return body


def _sc_scatter(yl, yr, src3, dst3, zeros_bd, n, d, e):
  return _sc_scatter_kernel(n, d, e)(yl, yr, src3, dst3, zeros_bd)


def _dinv_col(deg_ref, n):
  d = lax.rsqrt(deg_ref[0, :n] + deg_ref[1, :n] + 1.0)
  return d.reshape(n, 1)


def _tc_h0_body(x_ref, w_ref, b_ref, o_ref):
  h = jnp.dot(x_ref[...], w_ref[...], preferred_element_type=jnp.float32)
  o_ref[...] = jnp.maximum(h + b_ref[...], 0.0)


def _tc_y_body(n, h_ref, w_ref, deg_ref, ol_ref, or_ref):
  dh = ol_ref.shape[1]
  dinv = _dinv_col(deg_ref, n)
  xw = jnp.dot(h_ref[...], w_ref[...], preferred_element_type=jnp.float32)
  y = xw * dinv
  ol_ref[...] = y[:, :dh]
  or_ref[...] = y[:, dh:]


def _tc_layer_body(n, p_ref, yl_ref, yr_ref, deg_ref, b_ref, w_ref,
                   ol_ref, or_ref):
  dh = ol_ref.shape[1]
  dinv = _dinv_col(deg_ref, n)
  p = jnp.concatenate([p_ref[0, :n], p_ref[1, :n]], axis=1)
  y = jnp.concatenate([yl_ref[...], yr_ref[...]], axis=1)
  h = jnp.maximum(dinv * (p + y) + b_ref[...], 0.0)
  xw = jnp.dot(h, w_ref[...], preferred_element_type=jnp.float32)
  yn = xw * dinv
  ol_ref[...] = yn[:, :dh]
  or_ref[...] = yn[:, dh:]


def _tc_final_body(n, p_ref, yl_ref, yr_ref, deg_ref, b_ref, g_ref, bt_ref,
                   o_ref):
  dinv = _dinv_col(deg_ref, n)
  p = jnp.concatenate([p_ref[0, :n], p_ref[1, :n]], axis=1)
  y = jnp.concatenate([yl_ref[...], yr_ref[...]], axis=1)
  h = jnp.maximum(dinv * (p + y) + b_ref[...], 0.0)
  mean = jnp.mean(h, axis=0, keepdims=True)
  cen = h - mean
  var = jnp.mean(cen * cen, axis=0, keepdims=True)
  o_ref[...] = cen * lax.rsqrt(var + 1e-5) * g_ref[...] + bt_ref[...]


def _tc_call(body, out_shape, *args):
  return pl.pallas_call(body, out_shape=out_shape)(*args)


def kernel(x, edge_index, batch, W0, b0, Wc1, bc1, Wc2, bc2, Wc3, bc3,
           gamma, beta):
  n, d = x.shape
  dh = d // 2
  e = edge_index.shape[1]
  npad = _npad(n)
  nw = NC * NS
  kper_w = e // (nw * CHUNK)
  dst3w = edge_index[1].reshape(nw, kper_w, CHUNK)
  q = NB * CHUNK2
  perp = -(-(e // NS) // q) * q
  ep = NS * perp
  npad_ids = jnp.arange(ep - e, dtype=jnp.int32)
  src_p = jnp.concatenate([edge_index[0], npad_ids % n])
  dst_p = jnp.concatenate([edge_index[1], n + npad_ids % (npad - n)])
  src3 = src_p.reshape(NS, perp // q, NB, CHUNK2)
  dst3 = dst_p.reshape(NS, perp // q, NB, CHUNK2)
  zeros_np = jnp.zeros((npad,), jnp.float32)
  zeros_bd = jnp.zeros((npad // NS // 4, dh), jnp.float32)
  f32 = jnp.float32

  degp = _sc_deg(dst3w, zeros_np, n, e).reshape(NC, npad)
  yhalf = jax.ShapeDtypeStruct((n, dh), f32)
  h0 = _tc_call(_tc_h0_body, jax.ShapeDtypeStruct((n, d), f32),
                x, W0, b0.reshape(1, d))
  yl, yr = _tc_call(functools.partial(_tc_y_body, n), [yhalf, yhalf],
                    h0, Wc1, degp)

  for (w_next, b_cur) in ((Wc2, bc1), (Wc3, bc2)):
    p = _sc_scatter(yl, yr, src3, dst3, zeros_bd, n, d, ep)
    yl, yr = _tc_call(functools.partial(_tc_layer_body, n), [yhalf, yhalf],
                      p, yl, yr, degp, b_cur.reshape(1, d), w_next)

  p = _sc_scatter(yl, yr, src3, dst3, zeros_bd, n, d, ep)
  out = _tc_call(functools.partial(_tc_final_body, n),
                 jax.ShapeDtypeStruct((n, d), f32),
                 p, yl, yr, degp, bc3.reshape(1, d),
                 gamma.reshape(1, d), beta.reshape(1, d))
  return out
